# Initial kernel scaffold; baseline (speedup 1.0000x reference)
import jax
import jax.numpy as jnp
from jax.experimental import pallas as pl


def kernel(points, conv1_w, bn1_gamma, bn1_beta, conv2_w, bn2_gamma, bn2_beta):
    raise NotImplementedError("write your pallas kernel here")


# speedup vs baseline: 1.1589x; 1.1589x over previous
import functools

import jax
import jax.numpy as jnp
import numpy as np
from jax.experimental import pallas as pl
from jax.experimental.pallas import tpu as pltpu

IN_CH = 4
OUT_CH = 64
T = 100
MAX_PILLARS = 12000
PX = 0.16
PY = 0.16
X_MIN, Y_MIN, Z_MIN, X_MAX, Y_MAX, Z_MAX = -40.0, -40.0, -3.0, 40.0, 40.0, 1.0
NX = int(round((X_MAX - X_MIN) / PX))
NY = int(round((Y_MAX - Y_MIN) / PY))
EPS = 1e-5
BIG = NX * NY

_INT_DT = jnp.asarray(np.zeros((), dtype=np.int64)).dtype

M = MAX_PILLARS * T
P_TILE = 20
R_TILE = P_TILE * T
N_TILES = MAX_PILLARS // P_TILE


def _row_mask(i, p_scalar, n, scale):
    idx = jax.lax.broadcasted_iota(jnp.int32, (n, 1), 0) + i * n
    return idx < p_scalar * scale


def _h1(q_ref, c2_ref, wq_ref, wc_ref):
    q = q_ref[0]
    h1 = jnp.dot(q, wq_ref[...], preferred_element_type=jnp.float32)
    rows = jax.lax.broadcasted_iota(jnp.int32, (R_TILE, P_TILE), 0) // T
    cols = jax.lax.broadcasted_iota(jnp.int32, (R_TILE, P_TILE), 1)
    expand = (rows == cols).astype(jnp.float32)
    cflat = jnp.dot(expand, c2_ref[0, 0], preferred_element_type=jnp.float32)
    h1 = h1 + jnp.dot(cflat, wc_ref[...], preferred_element_type=jnp.float32)
    return h1


def _pass1(q_ref, c2_ref, p_ref, wq_ref, wc_ref, sum_ref, sq_ref):
    i = pl.program_id(1)
    h1 = _h1(q_ref, c2_ref, wq_ref, wc_ref)
    mask = _row_mask(i, p_ref[pl.program_id(0), 0], R_TILE, T)
    hm = jnp.where(mask, h1, 0.0)
    s = jnp.sum(hm, axis=0, keepdims=True)
    s2 = jnp.sum(hm * h1, axis=0, keepdims=True)

    @pl.when(i == 0)
    def _():
        sum_ref[0] = s
        sq_ref[0] = s2

    @pl.when(i != 0)
    def _():
        sum_ref[0] += s
        sq_ref[0] += s2


def _pass2(q_ref, c2_ref, p_ref, wq_ref, wc_ref, s1_ref, t1_ref, w2_ref,
           sum_ref, sq_ref):
    i = pl.program_id(1)
    h1 = _h1(q_ref, c2_ref, wq_ref, wc_ref)
    g = jnp.maximum(h1 * s1_ref[0] + t1_ref[0], 0.0)
    h2 = jnp.dot(g, w2_ref[...], preferred_element_type=jnp.float32)
    mask = _row_mask(i, p_ref[pl.program_id(0), 0], R_TILE, T)
    hm = jnp.where(mask, h2, 0.0)
    s = jnp.sum(hm, axis=0, keepdims=True)
    s2 = jnp.sum(hm * h2, axis=0, keepdims=True)

    @pl.when(i == 0)
    def _():
        sum_ref[0] = s
        sq_ref[0] = s2

    @pl.when(i != 0)
    def _():
        sum_ref[0] += s
        sq_ref[0] += s2


def _pass3(q_ref, c2_ref, p_ref, wq_ref, wc_ref, s1_ref, t1_ref, w2_ref,
           s2s_ref, t2s_ref, out_ref):
    i = pl.program_id(1)
    h1 = _h1(q_ref, c2_ref, wq_ref, wc_ref)
    g = jnp.maximum(h1 * s1_ref[0] + t1_ref[0], 0.0)
    h2 = jnp.dot(g, w2_ref[...], preferred_element_type=jnp.float32)
    o = jnp.maximum(h2 * s2s_ref[0] + t2s_ref[0], 0.0)
    o3 = o.reshape(P_TILE, T, OUT_CH)
    feats = jnp.max(o3, axis=1)
    pmask = _row_mask(i, p_ref[pl.program_id(0), 0], P_TILE, 1)
    out_ref[0, 0] = jnp.where(pmask, feats, 0.0)


def _bin_one(pts):
    n = pts.shape[0]
    x, y, z = pts[:, 0], pts[:, 1], pts[:, 2]
    valid = ((x >= X_MIN) & (x < X_MAX)
             & (y >= Y_MIN) & (y < Y_MAX)
             & (z >= Z_MIN) & (z < Z_MAX))
    xi = jnp.clip(jnp.floor((x - X_MIN) / PX).astype(jnp.int32), 0, NX - 1)
    yi = jnp.clip(jnp.floor((y - Y_MIN) / PY).astype(jnp.int32), 0, NY - 1)
    pkey = jnp.where(valid, xi * NY + yi, BIG)
    skeys = jnp.sort(pkey)
    new = jnp.concatenate([jnp.ones((1,), dtype=bool), skeys[1:] != skeys[:-1]])
    new_valid = new & (skeys < BIG)
    ranks = jnp.cumsum(new_valid.astype(jnp.int32)) - 1
    p_cnt = jnp.minimum(ranks[-1] + 1, MAX_PILLARS)
    uk_idx = jnp.where(new_valid & (ranks < MAX_PILLARS), ranks, MAX_PILLARS)
    uk = jnp.zeros((MAX_PILLARS,), dtype=jnp.int32).at[uk_idx].set(
        skeys, mode='drop')
    inv = ranks[jnp.searchsorted(skeys, pkey, side='left')]
    keep = valid & (inv < MAX_PILLARS)
    sort_key = jnp.where(keep, inv, MAX_PILLARS)
    perm = jnp.argsort(sort_key, stable=True)
    s_inv = sort_key[perm]
    iota = jnp.arange(n, dtype=jnp.int32)
    new_g = jnp.concatenate([jnp.ones((1,), dtype=bool),
                             s_inv[1:] != s_inv[:-1]])
    starts = jax.lax.cummax(jnp.where(new_g, iota, 0))
    intra = iota - starts
    keep2 = (s_inv < MAX_PILLARS) & (intra < T)
    row = jnp.where(keep2, s_inv, MAX_PILLARS)
    slot = jnp.where(keep2, intra, T)
    sel = pts[perm]
    pillar = jnp.zeros((MAX_PILLARS, T, IN_CH), dtype=jnp.float32
                       ).at[row, slot].set(sel, mode='drop')
    return pillar.reshape(M, IN_CH), uk, p_cnt


def kernel(points, conv1_w, bn1_gamma, bn1_beta, conv2_w, bn2_gamma, bn2_beta):
    bsz = points.shape[0]
    q, uk, p_cnt = jax.vmap(_bin_one)(points)
    p_cnt = p_cnt.astype(jnp.int32)

    xi_u = (uk // NY).astype(jnp.float32)
    yi_u = (uk % NY).astype(jnp.float32)
    xc = xi_u * PX + X_MIN + PX / 2.0
    yc = yi_u * PY + Y_MIN + PY / 2.0
    c2 = jnp.stack([xc, yc], axis=-1).reshape(bsz, N_TILES, P_TILE, 2)

    w1 = conv1_w
    wq = jnp.concatenate([
        (w1[:, 0] + w1[:, 6])[:, None],
        (w1[:, 1] + w1[:, 7])[:, None],
        w1[:, 2][:, None],
        w1[:, 3][:, None],
    ], axis=1).T
    wc = jnp.stack([w1[:, 4] - w1[:, 6], w1[:, 5] - w1[:, 7]], axis=0)
    w2t = conv2_w.T

    p_in = p_cnt[:, None]
    count = jnp.maximum(p_cnt * T, 1).astype(jnp.float32)[:, None]

    grid = (bsz, N_TILES)
    q_spec = pl.BlockSpec((1, R_TILE, IN_CH), lambda b, i: (b, i, 0))
    c2_spec = pl.BlockSpec((1, 1, P_TILE, 2), lambda b, i: (b, i, 0, 0))
    p_spec = pl.BlockSpec((bsz, 1), lambda b, i: (0, 0),
                          memory_space=pltpu.SMEM)
    wq_spec = pl.BlockSpec((IN_CH, OUT_CH), lambda b, i: (0, 0))
    wc_spec = pl.BlockSpec((2, OUT_CH), lambda b, i: (0, 0))
    w2_spec = pl.BlockSpec((OUT_CH, OUT_CH), lambda b, i: (0, 0))
    vec_spec = pl.BlockSpec((1, 1, OUT_CH), lambda b, i: (b, 0, 0))
    sum_shape = jax.ShapeDtypeStruct((bsz, 1, OUT_CH), jnp.float32)

    s1_, q1_ = pl.pallas_call(
        _pass1,
        grid=grid,
        in_specs=[q_spec, c2_spec, p_spec, wq_spec, wc_spec],
        out_specs=[vec_spec, vec_spec],
        out_shape=[sum_shape, sum_shape],
    )(q, c2, p_in, wq, wc)

    mean1 = s1_[:, 0] / count
    var1 = jnp.maximum(q1_[:, 0] / count - mean1 * mean1, 0.0)
    sc1 = bn1_gamma[None, :] / jnp.sqrt(var1 + EPS)
    sh1 = bn1_beta[None, :] - mean1 * sc1

    s2_, q2_ = pl.pallas_call(
        _pass2,
        grid=grid,
        in_specs=[q_spec, c2_spec, p_spec, wq_spec, wc_spec,
                  vec_spec, vec_spec, w2_spec],
        out_specs=[vec_spec, vec_spec],
        out_shape=[sum_shape, sum_shape],
    )(q, c2, p_in, wq, wc, sc1[:, None], sh1[:, None], w2t)

    mean2 = s2_[:, 0] / count
    var2 = jnp.maximum(q2_[:, 0] / count - mean2 * mean2, 0.0)
    sc2 = bn2_gamma[None, :] / jnp.sqrt(var2 + EPS)
    sh2 = bn2_beta[None, :] - mean2 * sc2

    feats = pl.pallas_call(
        _pass3,
        grid=grid,
        in_specs=[q_spec, c2_spec, p_spec, wq_spec, wc_spec,
                  vec_spec, vec_spec, w2_spec, vec_spec, vec_spec],
        out_specs=pl.BlockSpec((1, 1, P_TILE, OUT_CH),
                               lambda b, i: (b, i, 0, 0)),
        out_shape=jax.ShapeDtypeStruct((bsz, N_TILES, P_TILE, OUT_CH),
                                       jnp.float32),
    )(q, c2, p_in, wq, wc, sc1[:, None], sh1[:, None], w2t, sc2[:, None], sh2[:, None])
    feats = feats.reshape(bsz, MAX_PILLARS, OUT_CH)

    col0 = jnp.broadcast_to(
        jnp.arange(bsz, dtype=_INT_DT)[:, None], (bsz, MAX_PILLARS))
    coords = jnp.stack(
        [col0, (uk // NY).astype(_INT_DT), (uk % NY).astype(_INT_DT)],
        axis=2)
    return feats, coords
